# baseline (speedup 1.0000x reference)
import dataclasses

import jax
import jax.numpy as jnp
from jax import lax
from jax.experimental import pallas as pl
from jax.experimental.pallas import tpu as pltpu
from jax.experimental.pallas import tpu_sc as plsc

_USER_NUM = 1000000
_TABLE_ROWS = 2 * _USER_NUM
_B = 16384
_F = 16
_NC = 2
_NS = 16
_NW = _NC * _NS
_BPW = _B // _NW
_CHUNK = 128
_NCH = _BPW // _CHUNK
_LANES = 16
_GRP = _CHUNK // _LANES


def _fm_sc_body(user_ref, item_ref, emb_ref, lin_ref, bias_ref, out_ref,
                uidx, iidx, urow, irow, ulrow, ilrow,
                ubuf, ibuf, ulbuf, ilbuf, outv, biasv, sem):
    wid = lax.axis_index("s") * _NC + lax.axis_index("c")
    row0 = wid * _NCH
    base = wid * _BPW

    pltpu.sync_copy(user_ref.at[pl.ds(row0, _NCH)], uidx)
    pltpu.sync_copy(item_ref.at[pl.ds(row0, _NCH)], iidx)
    pltpu.sync_copy(bias_ref, biasv)

    for j in range(_NCH):
        for c in range(_CHUNK // _LANES):
            sl = (j, pl.ds(c * _LANES, _LANES))
            iv = iidx[sl] + _USER_NUM
            iidx[sl] = iv
            uv = uidx[sl]
            urow[sl] = uv >> 3
            irow[sl] = iv >> 3
            ulrow[sl] = uv >> 7
            ilrow[sl] = iv >> 7

    b = biasv[...]

    for j in range(_NCH):
        cps = (pltpu.async_copy(emb_ref.at[urow.at[j]], ubuf, sem),
               pltpu.async_copy(emb_ref.at[irow.at[j]], ibuf, sem),
               pltpu.async_copy(lin_ref.at[ulrow.at[j]], ulbuf, sem),
               pltpu.async_copy(lin_ref.at[ilrow.at[j]], ilbuf, sem))
        for cp in cps:
            cp.wait()

        @pl.loop(0, _GRP)
        def _(g):
            lanes = jnp.arange(_LANES, dtype=jnp.int32)
            rsel = g * _LANES + lanes
            isl = (j, pl.ds(g * _LANES, _LANES))
            uv = uidx[isl]
            iv = iidx[isl]
            uoff = (uv & 7) << 4
            ioff = (iv & 7) << 4
            acc = jnp.zeros((_LANES,), jnp.float32)
            for f in range(_F):
                a = plsc.load_gather(ubuf, [rsel, uoff + f])
                c = plsc.load_gather(ibuf, [rsel, ioff + f])
                acc = acc + a * c
            ul = plsc.load_gather(ulbuf, [rsel, uv & 127])
            il = plsc.load_gather(ilbuf, [rsel, iv & 127])
            outv[pl.ds(j * _CHUNK + g * _LANES, _LANES)] = acc + ul + il + b

    pltpu.sync_copy(outv, out_ref.at[pl.ds(base, _BPW)])


def kernel(user, item, emb_table, lin_table, bias):
    user2 = user.reshape(_NW * _NCH, _CHUNK)
    item2 = item.reshape(_NW * _NCH, _CHUNK)
    emb2 = emb_table.reshape(_TABLE_ROWS // 8, 128)
    lin2 = lin_table.reshape(_TABLE_ROWS // 128, 128)
    bias16 = jnp.broadcast_to(bias, (_LANES,))
    mesh = plsc.VectorSubcoreMesh(core_axis_name="c", subcore_axis_name="s")
    cp = pltpu.CompilerParams()
    if "needs_layout_passes" in pltpu.CompilerParams.__dataclass_fields__:
        cp = dataclasses.replace(cp, needs_layout_passes=False)
    f = pl.kernel(
        _fm_sc_body,
        out_type=jax.ShapeDtypeStruct((_B,), jnp.float32),
        mesh=mesh,
        scratch_types=[
            pltpu.VMEM((_NCH, _CHUNK), jnp.int32),
            pltpu.VMEM((_NCH, _CHUNK), jnp.int32),
            pltpu.VMEM((_NCH, _CHUNK), jnp.int32),
            pltpu.VMEM((_NCH, _CHUNK), jnp.int32),
            pltpu.VMEM((_NCH, _CHUNK), jnp.int32),
            pltpu.VMEM((_NCH, _CHUNK), jnp.int32),
            pltpu.VMEM((_CHUNK, 128), jnp.float32),
            pltpu.VMEM((_CHUNK, 128), jnp.float32),
            pltpu.VMEM((_CHUNK, 128), jnp.float32),
            pltpu.VMEM((_CHUNK, 128), jnp.float32),
            pltpu.VMEM((_BPW,), jnp.float32),
            pltpu.VMEM((_LANES,), jnp.float32),
            pltpu.SemaphoreType.DMA,
        ],
        compiler_params=cp,
    )
    return f(user2, item2, emb2, lin2, bias16)
